# Initial kernel scaffold; baseline (speedup 1.0000x reference)
import jax
import jax.numpy as jnp
from jax.experimental import pallas as pl


def kernel(logits, labels):
    raise NotImplementedError("write your pallas kernel here")


# speedup vs baseline: 5.7881x; 5.7881x over previous
import functools

import jax
import jax.numpy as jnp
from jax import lax
from jax.experimental import pallas as pl
from jax.experimental.pallas import tpu as pltpu
from jax.experimental.pallas import tpu_sc as plsc

N = 1_000_000
C = 10
NBINS = 15

CHUNK = 2048
NCHUNK = 163
PART = CHUNK * NCHUNK
NPAD = 3 * PART
PAD_VAL = 1.96875

B1 = 4096
SUB2 = 2048
NR = 30
NO_SLOT = 30


def _wid():
    return lax.axis_index("s") * 2 + lax.axis_index("c")


BLK = 2048


def _softmax_body(x_ref, o_ref):
    x = x_ref[...]
    m = jnp.max(x, axis=1, keepdims=True)
    e = jnp.exp(x - m)
    p = e / jnp.sum(e, axis=1, keepdims=True)
    col = pl.program_id(0) * BLK + lax.broadcasted_iota(jnp.int32, (C, BLK), 1)
    vals = jnp.where(col < N, p.T, PAD_VAL)
    o_ref[...] = lax.bitcast_convert_type(vals, jnp.int32)


def _softmax_t(logits):
    logits_p = jnp.pad(logits, ((0, NPAD - N), (0, 0)))
    return pl.pallas_call(
        _softmax_body,
        grid=(NPAD // BLK,),
        in_specs=[pl.BlockSpec((BLK, C), lambda i: (i, 0))],
        out_specs=pl.BlockSpec((C, BLK), lambda i: (0, i)),
        out_shape=jax.ShapeDtypeStruct((C, NPAD), jnp.int32),
    )(logits_p)


@functools.lru_cache(maxsize=1)
def _sc_kernels():
    mesh = plsc.VectorSubcoreMesh(core_axis_name="c", subcore_axis_name="s")

    @functools.partial(
        pl.kernel,
        out_type=jax.ShapeDtypeStruct((30, B1 * 16), jnp.int32),
        mesh=mesh,
        compiler_params=pltpu.CompilerParams(needs_layout_passes=False),
        scratch_types=[
            pltpu.VMEM((CHUNK,), jnp.int32),
            pltpu.VMEM((B1 * 16,), jnp.int32),
        ],
    )
    def sc_hist1(keys_hbm, out_hbm, buf, hist):
        wid = _wid()

        @pl.when(wid < 30)
        def _():
            cls = wid // 3
            col0 = (wid % 3) * PART
            lane = lax.iota(jnp.int32, 16)
            ones = jnp.ones((16,), jnp.int32)

            def zero_body(i, _):
                hist[pl.ds(i * 16, 16)] = jnp.zeros((16,), jnp.int32)
                return 0

            lax.fori_loop(0, B1, zero_body, 0)

            def chunk_body(k, _):
                pltpu.sync_copy(keys_hbm.at[cls, pl.ds(col0 + k * CHUNK, CHUNK)], buf)

                def vec_body(i, _):
                    key = buf[pl.ds(i * 16, 16)]
                    p = jnp.minimum(lax.shift_right_logical(key, 18), B1 - 1)
                    plsc.addupdate_scatter(hist, [p * 16 + lane], ones)
                    return 0

                lax.fori_loop(0, CHUNK // 16, vec_body, 0)
                return 0

            lax.fori_loop(0, NCHUNK, chunk_body, 0)
            pltpu.sync_copy(hist, out_hbm.at[wid])

    @functools.partial(
        pl.kernel,
        out_type=jax.ShapeDtypeStruct((30, NR * SUB2), jnp.int32),
        mesh=mesh,
        compiler_params=pltpu.CompilerParams(needs_layout_passes=False),
        scratch_types=[
            pltpu.VMEM((CHUNK,), jnp.int32),
            pltpu.VMEM((B1,), jnp.int32),
            pltpu.VMEM((NR * SUB2,), jnp.int32),
        ],
    )
    def sc_hist2(keys_hbm, tbl_hbm, out_hbm, buf, tbl, hist):
        wid = _wid()

        @pl.when(wid < 30)
        def _():
            cls = wid // 3
            col0 = (wid % 3) * PART
            ones = jnp.ones((16,), jnp.int32)
            pltpu.sync_copy(tbl_hbm.at[cls], tbl)

            def zero_body(i, _):
                hist[pl.ds(i * 16, 16)] = jnp.zeros((16,), jnp.int32)
                return 0

            lax.fori_loop(0, NR * SUB2 // 16, zero_body, 0)

            def chunk_body(k, _):
                pltpu.sync_copy(keys_hbm.at[cls, pl.ds(col0 + k * CHUNK, CHUNK)], buf)

                def vec_body(i, _):
                    key = buf[pl.ds(i * 16, 16)]
                    p = jnp.minimum(lax.shift_right_logical(key, 18), B1 - 1)
                    sl = plsc.load_gather(tbl, [p])
                    valid = sl < NO_SLOT
                    s2 = jnp.bitwise_and(lax.shift_right_logical(key, 7), SUB2 - 1)
                    idx = jnp.where(valid, sl * SUB2 + s2, 0)
                    plsc.addupdate_scatter(hist, [idx], ones, mask=valid)
                    return 0

                lax.fori_loop(0, CHUNK // 16, vec_body, 0)
                return 0

            lax.fori_loop(0, NCHUNK, chunk_body, 0)
            pltpu.sync_copy(hist, out_hbm.at[wid])

    @functools.partial(
        pl.kernel,
        out_type=jax.ShapeDtypeStruct((30, 1536), jnp.float32),
        mesh=mesh,
        compiler_params=pltpu.CompilerParams(needs_layout_passes=False),
        scratch_types=[
            pltpu.VMEM((CHUNK,), jnp.float32),
            pltpu.VMEM((CHUNK,), jnp.int32),
            pltpu.VMEM((16,), jnp.float32),
            pltpu.VMEM((1536,), jnp.float32),
        ],
    )
    def sc_bins(conf_hbm, lab_hbm, edges_hbm, out_hbm, bufc, bufl, ev, acc):
        wid = _wid()

        @pl.when(wid < 30)
        def _():
            cls = wid // 3
            col0 = (wid % 3) * PART
            lane = lax.iota(jnp.int32, 16)
            onesf = jnp.ones((16,), jnp.float32)
            pltpu.sync_copy(edges_hbm.at[cls], ev)
            e15v = plsc.load_gather(ev, [jnp.full((16,), 15, jnp.int32)])
            clsv = jnp.ones((16,), jnp.int32) * cls

            def zero_body(i, _):
                acc[pl.ds(i * 16, 16)] = jnp.zeros((16,), jnp.float32)
                return 0

            lax.fori_loop(0, 1536 // 16, zero_body, 0)

            def chunk_body(k, _):
                pltpu.sync_copy(conf_hbm.at[cls, pl.ds(col0 + k * CHUNK, CHUNK)], bufc)
                pltpu.sync_copy(lab_hbm.at[pl.ds(col0 + k * CHUNK, CHUNK)], bufl)

                def vec_body(i, _):
                    conf = bufc[pl.ds(i * 16, 16)]
                    labv = bufl[pl.ds(i * 16, 16)]
                    m = jnp.zeros((16,), jnp.int32)
                    for step in (8, 4, 2, 1):
                        cand = m + step
                        ec = plsc.load_gather(ev, [cand - 1])
                        m = jnp.where(ec < conf, cand, m)
                    labi = jnp.where(labv == clsv, 1, 0)
                    base = (m * 2 + labi) * 48 + lane
                    valid = conf <= e15v
                    plsc.addupdate_scatter(acc, [base], onesf, mask=valid)
                    plsc.addupdate_scatter(acc, [base + 16], conf, mask=valid)
                    plsc.addupdate_scatter(acc, [base + 32], conf * conf, mask=valid)
                    return 0

                lax.fori_loop(0, CHUNK // 16, vec_body, 0)
                return 0

            lax.fori_loop(0, NCHUNK, chunk_body, 0)
            pltpu.sync_copy(acc, out_hbm.at[wid])

    return sc_hist1, sc_hist2, sc_bins


def _locate(hist, rank):
    cum = jnp.cumsum(hist, axis=-1)
    mask = cum[..., None, :] <= rank[..., None]
    b = mask.sum(-1).astype(jnp.int32)
    before = (mask * hist[..., None, :]).sum(-1)
    return b, rank - before


def kernel(logits, labels):
    sc_hist1, sc_hist2, sc_bins = _sc_kernels()
    keysp = _softmax_t(logits)
    confp = lax.bitcast_convert_type(keysp, jnp.float32)
    labp = jnp.pad(labels.astype(jnp.int32), (0, NPAD - N))

    xq = jnp.linspace(0.0, float(N), NBINS + 1)
    kf = jnp.floor(xq[1:NBINS])
    kj = kf.astype(jnp.int32)
    frac = (xq[1:NBINS] - kf).astype(jnp.float32)
    ranks = jnp.concatenate([
        jnp.zeros((1,), jnp.int32),
        jnp.stack([kj, kj + 1], axis=1).reshape(-1),
        jnp.full((1,), N - 1, jnp.int32),
    ])

    h1 = sc_hist1(keysp)
    hist = h1.reshape(C, 3, B1, 16).sum((1, 3))
    rk = jnp.broadcast_to(ranks, (C, NR))
    b, lr1 = _locate(hist, rk)

    ii = jnp.arange(NR, dtype=jnp.int32)
    same = b[:, None, :] == b[:, :, None]
    slot = jnp.min(jnp.where(same, ii[None, None, :], NO_SLOT), axis=-1)
    slot = slot.astype(jnp.int32)
    pgrid = jnp.arange(B1, dtype=jnp.int32)
    tbl = jnp.min(
        jnp.where(b[:, :, None] == pgrid[None, None, :], ii[None, :, None], NO_SLOT),
        axis=1,
    ).astype(jnp.int32)

    h2 = sc_hist2(keysp, tbl)
    h2c = h2.reshape(C, 3, NR, SUB2).sum(1)
    hsel = jnp.take_along_axis(h2c, slot[:, :, None], axis=1)
    s2, lr2 = _locate(hsel, lr1[:, :, None])
    s2, lr2 = s2[..., 0], lr2[..., 0]

    cw = jnp.take_along_axis(hsel, s2[:, :, None], axis=-1)[..., 0]
    pos = jnp.clip(
        jnp.round(128.0 * (lr2.astype(jnp.float32) + 0.5)
                  / jnp.maximum(cw.astype(jnp.float32), 1.0)).astype(jnp.int32),
        0, 127)
    vkey = b * (1 << 18) + s2 * 128 + pos
    v = lax.bitcast_convert_type(vkey, jnp.float32)

    va = v[:, 1:29:2]
    vb = v[:, 2:30:2]
    emid = va + frac[None, :] * (vb - va)
    edges = jnp.concatenate([v[:, :1], emid, v[:, 29:30]], axis=1)

    st = sc_bins(confp, labp, edges)
    s = st.reshape(C, 3, 16, 2, 3, 16).sum((1, 5))
    s = s[:, 1:16]
    cnt0, cnt1 = s[..., 0, 0], s[..., 1, 0]
    sc0, sc1 = s[..., 0, 1], s[..., 1, 1]
    ss2 = s[..., 0, 2] + s[..., 1, 2]
    cnt = cnt0 + cnt1
    denom = jnp.maximum(cnt - 1.0, 1.0)
    a0 = cnt1 / denom
    a1 = (cnt1 - 1.0) / denom
    term = ss2 - 2.0 * (a0 * sc0 + a1 * sc1) + cnt0 * a0 * a0 + cnt1 * a1 * a1
    term = jnp.where(cnt >= 2.0, term, 0.0)
    return jnp.mean(term.sum(1)) / N


# speedup vs baseline: 6.3587x; 1.0986x over previous
import functools

import jax
import jax.numpy as jnp
from jax import lax
from jax.experimental import pallas as pl
from jax.experimental.pallas import tpu as pltpu
from jax.experimental.pallas import tpu_sc as plsc

N = 1_000_000
C = 10
NBINS = 15

CHUNK = 8192
NCHUNK = 41
PART = CHUNK * NCHUNK
NPAD = 3 * PART
PAD_VAL = 1.96875

B1 = 4096
SUB2 = 2048
NR = 30
NO_SLOT = 30


def _wid():
    return lax.axis_index("s") * 2 + lax.axis_index("c")


BLK = 2048


def _softmax_body(x_ref, o_ref):
    x = x_ref[...]
    m = jnp.max(x, axis=1, keepdims=True)
    e = jnp.exp(x - m)
    p = e / jnp.sum(e, axis=1, keepdims=True)
    col = pl.program_id(0) * BLK + lax.broadcasted_iota(jnp.int32, (C, BLK), 1)
    vals = jnp.where(col < N, p.T, PAD_VAL)
    o_ref[...] = lax.bitcast_convert_type(vals, jnp.int32)


def _softmax_t(logits):
    logits_p = jnp.pad(logits, ((0, NPAD - N), (0, 0)))
    return pl.pallas_call(
        _softmax_body,
        grid=(NPAD // BLK,),
        in_specs=[pl.BlockSpec((BLK, C), lambda i: (i, 0))],
        out_specs=pl.BlockSpec((C, BLK), lambda i: (0, i)),
        out_shape=jax.ShapeDtypeStruct((C, NPAD), jnp.int32),
    )(logits_p)


@functools.lru_cache(maxsize=1)
def _sc_kernels():
    mesh = plsc.VectorSubcoreMesh(core_axis_name="c", subcore_axis_name="s")

    @functools.partial(
        pl.kernel,
        out_type=jax.ShapeDtypeStruct((30, B1 * 16), jnp.int32),
        mesh=mesh,
        compiler_params=pltpu.CompilerParams(needs_layout_passes=False),
        scratch_types=[
            pltpu.VMEM((CHUNK,), jnp.int32),
            pltpu.VMEM((B1 * 16,), jnp.int32),
        ],
    )
    def sc_hist1(keys_hbm, out_hbm, buf, hist):
        wid = _wid()

        @pl.when(wid < 30)
        def _():
            cls = wid // 3
            col0 = (wid % 3) * PART
            lane = lax.iota(jnp.int32, 16)
            ones = jnp.ones((16,), jnp.int32)

            def zero_body(i, _):
                hist[pl.ds(i * 16, 16)] = jnp.zeros((16,), jnp.int32)
                return 0

            lax.fori_loop(0, B1, zero_body, 0)

            def chunk_body(k, _):
                pltpu.sync_copy(keys_hbm.at[cls, pl.ds(col0 + k * CHUNK, CHUNK)], buf)

                def vec_body(i, _):
                    key = buf[pl.ds(i * 16, 16)]
                    p = jnp.minimum(lax.shift_right_logical(key, 18), B1 - 1)
                    plsc.addupdate_scatter(hist, [p * 16 + lane], ones)
                    return 0

                lax.fori_loop(0, CHUNK // 16, vec_body, 0)
                return 0

            lax.fori_loop(0, NCHUNK, chunk_body, 0)
            pltpu.sync_copy(hist, out_hbm.at[wid])

    @functools.partial(
        pl.kernel,
        out_type=jax.ShapeDtypeStruct((30, NR * SUB2), jnp.int32),
        mesh=mesh,
        compiler_params=pltpu.CompilerParams(needs_layout_passes=False),
        scratch_types=[
            pltpu.VMEM((CHUNK,), jnp.int32),
            pltpu.VMEM((B1,), jnp.int32),
            pltpu.VMEM((NR * SUB2,), jnp.int32),
        ],
    )
    def sc_hist2(keys_hbm, tbl_hbm, out_hbm, buf, tbl, hist):
        wid = _wid()

        @pl.when(wid < 30)
        def _():
            cls = wid // 3
            col0 = (wid % 3) * PART
            ones = jnp.ones((16,), jnp.int32)
            pltpu.sync_copy(tbl_hbm.at[cls], tbl)

            def zero_body(i, _):
                hist[pl.ds(i * 16, 16)] = jnp.zeros((16,), jnp.int32)
                return 0

            lax.fori_loop(0, NR * SUB2 // 16, zero_body, 0)

            def chunk_body(k, _):
                pltpu.sync_copy(keys_hbm.at[cls, pl.ds(col0 + k * CHUNK, CHUNK)], buf)

                def vec_body(i, _):
                    key = buf[pl.ds(i * 16, 16)]
                    p = jnp.minimum(lax.shift_right_logical(key, 18), B1 - 1)
                    sl = plsc.load_gather(tbl, [p])
                    valid = sl < NO_SLOT
                    s2 = jnp.bitwise_and(lax.shift_right_logical(key, 7), SUB2 - 1)
                    idx = jnp.where(valid, sl * SUB2 + s2, 0)
                    plsc.addupdate_scatter(hist, [idx], ones, mask=valid)
                    return 0

                lax.fori_loop(0, CHUNK // 16, vec_body, 0)
                return 0

            lax.fori_loop(0, NCHUNK, chunk_body, 0)
            pltpu.sync_copy(hist, out_hbm.at[wid])

    @functools.partial(
        pl.kernel,
        out_type=jax.ShapeDtypeStruct((30, 1536), jnp.float32),
        mesh=mesh,
        compiler_params=pltpu.CompilerParams(needs_layout_passes=False),
        scratch_types=[
            pltpu.VMEM((CHUNK,), jnp.float32),
            pltpu.VMEM((CHUNK,), jnp.int32),
            pltpu.VMEM((16,), jnp.float32),
            pltpu.VMEM((1536,), jnp.float32),
        ],
    )
    def sc_bins(conf_hbm, lab_hbm, edges_hbm, out_hbm, bufc, bufl, ev, acc):
        wid = _wid()

        @pl.when(wid < 30)
        def _():
            cls = wid // 3
            col0 = (wid % 3) * PART
            lane = lax.iota(jnp.int32, 16)
            onesf = jnp.ones((16,), jnp.float32)
            pltpu.sync_copy(edges_hbm.at[cls], ev)
            e15v = plsc.load_gather(ev, [jnp.full((16,), 15, jnp.int32)])
            clsv = jnp.ones((16,), jnp.int32) * cls

            def zero_body(i, _):
                acc[pl.ds(i * 16, 16)] = jnp.zeros((16,), jnp.float32)
                return 0

            lax.fori_loop(0, 1536 // 16, zero_body, 0)

            def chunk_body(k, _):
                pltpu.sync_copy(conf_hbm.at[cls, pl.ds(col0 + k * CHUNK, CHUNK)], bufc)
                pltpu.sync_copy(lab_hbm.at[pl.ds(col0 + k * CHUNK, CHUNK)], bufl)

                def vec_body(i, _):
                    conf = bufc[pl.ds(i * 16, 16)]
                    labv = bufl[pl.ds(i * 16, 16)]
                    m = jnp.zeros((16,), jnp.int32)
                    for step in (8, 4, 2, 1):
                        cand = m + step
                        ec = plsc.load_gather(ev, [cand - 1])
                        m = jnp.where(ec < conf, cand, m)
                    labi = jnp.where(labv == clsv, 1, 0)
                    base = (m * 2 + labi) * 48 + lane
                    valid = conf <= e15v
                    plsc.addupdate_scatter(acc, [base], onesf, mask=valid)
                    plsc.addupdate_scatter(acc, [base + 16], conf, mask=valid)
                    plsc.addupdate_scatter(acc, [base + 32], conf * conf, mask=valid)
                    return 0

                lax.fori_loop(0, CHUNK // 16, vec_body, 0)
                return 0

            lax.fori_loop(0, NCHUNK, chunk_body, 0)
            pltpu.sync_copy(acc, out_hbm.at[wid])

    return sc_hist1, sc_hist2, sc_bins


def _locate(hist, rank):
    cum = jnp.cumsum(hist, axis=-1)
    mask = cum[..., None, :] <= rank[..., None]
    b = mask.sum(-1).astype(jnp.int32)
    before = (mask * hist[..., None, :]).sum(-1)
    return b, rank - before


def kernel(logits, labels):
    sc_hist1, sc_hist2, sc_bins = _sc_kernels()
    keysp = _softmax_t(logits)
    confp = lax.bitcast_convert_type(keysp, jnp.float32)
    labp = jnp.pad(labels.astype(jnp.int32), (0, NPAD - N))

    xq = jnp.linspace(0.0, float(N), NBINS + 1)
    kf = jnp.floor(xq[1:NBINS])
    kj = kf.astype(jnp.int32)
    frac = (xq[1:NBINS] - kf).astype(jnp.float32)
    ranks = jnp.concatenate([
        jnp.zeros((1,), jnp.int32),
        jnp.stack([kj, kj + 1], axis=1).reshape(-1),
        jnp.full((1,), N - 1, jnp.int32),
    ])

    h1 = sc_hist1(keysp)
    hist = h1.reshape(C, 3, B1, 16).sum((1, 3))
    rk = jnp.broadcast_to(ranks, (C, NR))
    b, lr1 = _locate(hist, rk)

    ii = jnp.arange(NR, dtype=jnp.int32)
    same = b[:, None, :] == b[:, :, None]
    slot = jnp.min(jnp.where(same, ii[None, None, :], NO_SLOT), axis=-1)
    slot = slot.astype(jnp.int32)
    pgrid = jnp.arange(B1, dtype=jnp.int32)
    tbl = jnp.min(
        jnp.where(b[:, :, None] == pgrid[None, None, :], ii[None, :, None], NO_SLOT),
        axis=1,
    ).astype(jnp.int32)

    h2 = sc_hist2(keysp, tbl)
    h2c = h2.reshape(C, 3, NR, SUB2).sum(1)
    hsel = jnp.take_along_axis(h2c, slot[:, :, None], axis=1)
    s2, lr2 = _locate(hsel, lr1[:, :, None])
    s2, lr2 = s2[..., 0], lr2[..., 0]

    cw = jnp.take_along_axis(hsel, s2[:, :, None], axis=-1)[..., 0]
    pos = jnp.clip(
        jnp.round(128.0 * (lr2.astype(jnp.float32) + 0.5)
                  / jnp.maximum(cw.astype(jnp.float32), 1.0)).astype(jnp.int32),
        0, 127)
    vkey = b * (1 << 18) + s2 * 128 + pos
    v = lax.bitcast_convert_type(vkey, jnp.float32)

    va = v[:, 1:29:2]
    vb = v[:, 2:30:2]
    emid = va + frac[None, :] * (vb - va)
    edges = jnp.concatenate([v[:, :1], emid, v[:, 29:30]], axis=1)

    st = sc_bins(confp, labp, edges)
    s = st.reshape(C, 3, 16, 2, 3, 16).sum((1, 5))
    s = s[:, 1:16]
    cnt0, cnt1 = s[..., 0, 0], s[..., 1, 0]
    sc0, sc1 = s[..., 0, 1], s[..., 1, 1]
    ss2 = s[..., 0, 2] + s[..., 1, 2]
    cnt = cnt0 + cnt1
    denom = jnp.maximum(cnt - 1.0, 1.0)
    a0 = cnt1 / denom
    a1 = (cnt1 - 1.0) / denom
    term = ss2 - 2.0 * (a0 * sc0 + a1 * sc1) + cnt0 * a0 * a0 + cnt1 * a1 * a1
    term = jnp.where(cnt >= 2.0, term, 0.0)
    return jnp.mean(term.sum(1)) / N
